# Initial kernel scaffold; baseline (speedup 1.0000x reference)
import jax
import jax.numpy as jnp
from jax.experimental import pallas as pl


def kernel(x_nchw, conv_w, gamma, beta):
    raise NotImplementedError("write your pallas kernel here")


# speedup vs baseline: 1.5570x; 1.5570x over previous
import jax
import jax.numpy as jnp
from jax.experimental import pallas as pl
from jax.experimental.pallas import tpu as pltpu

_VMEM_LIMIT = 48 * 1024 * 1024


def _conv_stats_kernel(x_ref, w_ref, y_ref, sum_ref, sqs_ref, *, k, ho, wo):
    slab = x_ref[0]
    rows = ho * wo
    cols = [slab[di:di + ho, dj:dj + wo, :].reshape(rows, -1)
            for di in range(k) for dj in range(k)]
    xcol = jnp.concatenate(cols, axis=1)
    acc = jnp.dot(xcol, w_ref[...],
                  preferred_element_type=jnp.float32)
    ones8 = jnp.ones((8, rows), jnp.float32)
    sum_ref[0] = jnp.dot(ones8, acc, preferred_element_type=jnp.float32)
    sqs_ref[0] = jnp.dot(ones8, acc * acc, preferred_element_type=jnp.float32)
    y_ref[0] = acc.T


def _bn_mish_kernel(y_ref, scale_ref, shift_ref, o_ref):
    z = y_ref[0] * scale_ref[...] + shift_ref[...]
    t = jnp.exp(jnp.minimum(z, 20.0))
    u = t * (t + 2.0)
    mish = z * u * pl.reciprocal(u + 2.0, approx=True)
    o_ref[0] = jnp.where(z > 20.0, z, mish).astype(o_ref.dtype)


def kernel(x_nchw, conv_w, gamma, beta):
    eps = 1e-5
    N, Cin, H, W = x_nchw.shape
    Cout, _, k, _ = conv_w.shape
    p = k // 2
    Ho, Wo = H, W
    rows = Ho * Wo

    x_nhwc = jnp.transpose(x_nchw, (0, 2, 3, 1)).astype(jnp.float32)
    xp = jnp.pad(x_nhwc, ((0, 0), (p, p), (p, p), (0, 0)))
    Hp, Wp = H + 2 * p, W + 2 * p

    w_flat = jnp.transpose(conv_w, (2, 3, 1, 0)).astype(jnp.float32)
    w_flat = w_flat.reshape(k * k * Cin, Cout)

    import functools
    conv_kernel = functools.partial(_conv_stats_kernel, k=k, ho=Ho, wo=Wo)
    y, sums, sqs = pl.pallas_call(
        conv_kernel,
        out_shape=(jax.ShapeDtypeStruct((N, Cout, rows), jnp.float32),
                   jax.ShapeDtypeStruct((N, 8, Cout), jnp.float32),
                   jax.ShapeDtypeStruct((N, 8, Cout), jnp.float32)),
        grid=(N,),
        in_specs=[pl.BlockSpec((1, Hp, Wp, Cin), lambda n: (n, 0, 0, 0)),
                  pl.BlockSpec((k * k * Cin, Cout), lambda n: (0, 0))],
        out_specs=(pl.BlockSpec((1, Cout, rows), lambda n: (n, 0, 0)),
                   pl.BlockSpec((1, 8, Cout), lambda n: (n, 0, 0)),
                   pl.BlockSpec((1, 8, Cout), lambda n: (n, 0, 0))),
        compiler_params=pltpu.CompilerParams(
            dimension_semantics=("parallel",),
            vmem_limit_bytes=_VMEM_LIMIT),
    )(xp, w_flat)

    count = N * rows
    s = jnp.sum(sums[:, 0, :], axis=0)
    q = jnp.sum(sqs[:, 0, :], axis=0)
    mean = s / count
    var = jnp.maximum(q / count - mean * mean, 0.0)
    inv_std = jax.lax.rsqrt(var + eps)
    g = gamma.astype(jnp.float32)
    scale = (g * inv_std).reshape(Cout, 1)
    shift = (beta.astype(jnp.float32) - mean * g * inv_std).reshape(Cout, 1)

    out_flat = pl.pallas_call(
        _bn_mish_kernel,
        out_shape=jax.ShapeDtypeStruct((N, Cout, rows), jnp.float32),
        grid=(N,),
        in_specs=[pl.BlockSpec((1, Cout, rows), lambda n: (n, 0, 0)),
                  pl.BlockSpec((Cout, 1), lambda n: (0, 0)),
                  pl.BlockSpec((Cout, 1), lambda n: (0, 0))],
        out_specs=pl.BlockSpec((1, Cout, rows), lambda n: (n, 0, 0)),
        compiler_params=pltpu.CompilerParams(
            dimension_semantics=("parallel",),
            vmem_limit_bytes=_VMEM_LIMIT),
    )(y, scale, shift)

    return out_flat.reshape(N, Cout, Ho, Wo)
